# baseline (speedup 1.0000x reference)
import functools

import jax
import jax.numpy as jnp
from jax import lax
from jax.experimental import pallas as pl
from jax.experimental.pallas import tpu as pltpu
from jax.experimental.pallas import tpu_sc as plsc

_TOPK = 8
_NUM_DRAFT = 48
_NEG_INF = float("-inf")
_BIG_I32 = 2**30
_W = 800
_CW = 20000
_Q = _CW // _W


def _stats_body(K, NC, x_ref, sm_ref, m0_ref, lse_ref):
    x = x_ref[0]
    m_pieces = []
    e_pieces = []
    for q in range(_Q):
        xq = x[:, :, q * _W:(q + 1) * _W]
        mq = jnp.max(xq, axis=2)
        eq = jnp.sum(jnp.exp(xq - mq[:, :, None]), axis=2)
        m_pieces.append(mq)
        e_pieces.append(eq)
    sm = jnp.concatenate(m_pieces, axis=1)
    sexp = jnp.concatenate(e_pieces, axis=1)
    m0 = jnp.max(sm, axis=1, keepdims=True)
    lse = jnp.log(jnp.sum(jnp.exp(sm - m0) * sexp, axis=1, keepdims=True))
    sm_ref[0] = sm
    m0_ref[0] = m0
    lse_ref[0] = lse


def _strip_select_body(S, NC, sm_ref, voff_ref, flat_ref):
    sm = sm_ref[...]
    R = sm.shape[0]
    lane_s = lax.broadcasted_iota(jnp.int32, (R, S), 1)
    q = lane_s // NC
    c = lane_s - q * NC
    voff = c * _CW + q * _W
    lane_k = lax.broadcasted_iota(jnp.int32, (R, _TOPK), 1)
    keep = jnp.zeros((R, S), jnp.bool_)
    for _ in range(_TOPK):
        m = jnp.max(sm, axis=1, keepdims=True)
        vsel = jnp.min(jnp.where(sm == m, voff, _BIG_I32), axis=1,
                       keepdims=True)
        hit = voff == vsel
        keep = jnp.logical_or(keep, hit)
        sm = jnp.where(hit, _NEG_INF, sm)
    voffs = jnp.zeros((R, _TOPK), jnp.int32)
    for k in range(_TOPK):
        vo_k = jnp.min(jnp.where(keep, voff, _BIG_I32), axis=1,
                       keepdims=True)
        keep = jnp.logical_and(keep, voff != vo_k)
        voffs = jnp.where(lane_k == k, vo_k, voffs)
    voff_ref[...] = voffs
    row_iota = lax.broadcasted_iota(jnp.int32, (R, _TOPK), 0)
    flat_ref[...] = row_iota * S + voffs // _W


def _make_sc_gather(n_idx, W, n_workers):
    chunk = n_idx // n_workers
    mesh = plsc.VectorSubcoreMesh(core_axis_name="c", subcore_axis_name="s")

    @functools.partial(
        pl.kernel,
        mesh=mesh,
        compiler_params=pltpu.CompilerParams(use_tc_tiling_on_sc=False),
        out_type=jax.ShapeDtypeStruct((n_idx, W), jnp.float32),
        scratch_types=[
            pltpu.VMEM((chunk,), jnp.int32),
            pltpu.VMEM((chunk, W), jnp.float32),
            pltpu.SemaphoreType.DMA,
        ],
    )
    def gather_sc(idx_hbm, table_hbm, out_hbm, idx_v, rows_v, sem):
        info = plsc.get_sparse_core_info()
        wid = lax.axis_index("s") * info.num_cores + lax.axis_index("c")
        base = wid * chunk
        pltpu.sync_copy(idx_hbm.at[pl.ds(base, chunk)], idx_v)
        pltpu.async_copy(table_hbm.at[idx_v], rows_v, sem).wait()
        pltpu.sync_copy(rows_v, out_hbm.at[pl.ds(base, chunk)])

    return gather_sc


def _row_topk_body(W, g_ref, voff_ref, m0_ref, lse_ref, scores_ref,
                   tokens_ref):
    g = g_ref[...]
    voffs = voff_ref[...]
    RB = g.shape[0]
    lane = lax.broadcasted_iota(jnp.int32, (1, 1, W), 2)
    gi = voffs[:, :, None] + lane
    lane_k = lax.broadcasted_iota(jnp.int32, (RB, _TOPK), 1)
    vals = jnp.zeros((RB, _TOPK), jnp.float32)
    toks = jnp.zeros((RB, _TOPK), jnp.int32)
    for i in range(_TOPK):
        m = jnp.max(g, axis=(1, 2), keepdims=True)
        jv = jnp.min(jnp.where(g == m, gi, _BIG_I32), axis=(1, 2),
                     keepdims=True)
        vals = jnp.where(lane_k == i, m[:, 0, :], vals)
        toks = jnp.where(lane_k == i, jv[:, 0, :], toks)
        g = jnp.where(gi == jv, _NEG_INF, g)
    scores_ref[...] = (vals - m0_ref[...]) - lse_ref[...]
    tokens_ref[...] = toks


def _merge_body(KK, scores_ref, tokens_ref, prev_ref, out_s_ref, out_t_ref):
    cum = scores_ref[...] + prev_ref[...]
    toks = tokens_ref[...]
    B = cum.shape[0]
    lane_kk = lax.broadcasted_iota(jnp.int32, (B, KK), 1)
    lane_t = lax.broadcasted_iota(jnp.int32, (B, _NUM_DRAFT), 1)

    def body(i, carry):
        cum, outv, outt = carry
        m = jnp.max(cum, axis=1, keepdims=True)
        jsel = jnp.min(jnp.where(cum == m, lane_kk, _BIG_I32), axis=1,
                       keepdims=True)
        tok = jnp.min(jnp.where(lane_kk == jsel, toks, _BIG_I32), axis=1,
                      keepdims=True)
        outv = jnp.where(lane_t == i, m, outv)
        outt = jnp.where(lane_t == i, tok, outt)
        cum = jnp.where(lane_kk == jsel, _NEG_INF, cum)
        return cum, outv, outt

    outv = jnp.zeros((B, _NUM_DRAFT), jnp.float32)
    outt = jnp.zeros((B, _NUM_DRAFT), jnp.int32)
    _, outv, outt = lax.fori_loop(0, _NUM_DRAFT, body, (cum, outv, outt))
    out_s_ref[...] = outv
    out_t_ref[...] = outt


def kernel(logits, prev_scores):
    B, K, V = logits.shape
    NC = V // _CW
    S = V // _W
    R = B * K

    x4 = logits.reshape(B, K, NC, _CW)
    sm, m0, lse = pl.pallas_call(
        functools.partial(_stats_body, K, NC),
        grid=(B,),
        in_specs=[pl.BlockSpec((1, K, NC, _CW), lambda i: (i, 0, 0, 0))],
        out_specs=[
            pl.BlockSpec((1, K, S), lambda i: (i, 0, 0)),
            pl.BlockSpec((1, K, 1), lambda i: (i, 0, 0)),
            pl.BlockSpec((1, K, 1), lambda i: (i, 0, 0)),
        ],
        out_shape=[
            jax.ShapeDtypeStruct((B, K, S), jnp.float32),
            jax.ShapeDtypeStruct((B, K, 1), jnp.float32),
            jax.ShapeDtypeStruct((B, K, 1), jnp.float32),
        ],
    )(x4)

    voffs, flat_ids = pl.pallas_call(
        functools.partial(_strip_select_body, S, NC),
        out_shape=[
            jax.ShapeDtypeStruct((R, _TOPK), jnp.int32),
            jax.ShapeDtypeStruct((R, _TOPK), jnp.int32),
        ],
    )(sm.reshape(R, S))

    n_idx = R * _TOPK
    gathered = _make_sc_gather(n_idx, _W, 32)(
        flat_ids.reshape(n_idx), logits.reshape(R * S, _W))

    RB = min(64, R)
    scores, tokens = pl.pallas_call(
        functools.partial(_row_topk_body, _W),
        grid=(R // RB,),
        in_specs=[
            pl.BlockSpec((RB, _TOPK, _W), lambda i: (i, 0, 0)),
            pl.BlockSpec((RB, _TOPK), lambda i: (i, 0)),
            pl.BlockSpec((RB, 1), lambda i: (i, 0)),
            pl.BlockSpec((RB, 1), lambda i: (i, 0)),
        ],
        out_specs=[
            pl.BlockSpec((RB, _TOPK), lambda i: (i, 0)),
            pl.BlockSpec((RB, _TOPK), lambda i: (i, 0)),
        ],
        out_shape=[
            jax.ShapeDtypeStruct((R, _TOPK), jnp.float32),
            jax.ShapeDtypeStruct((R, _TOPK), jnp.int32),
        ],
    )(gathered.reshape(R, _TOPK, _W), voffs, m0.reshape(R, 1),
      lse.reshape(R, 1))

    prev_rep = jnp.repeat(prev_scores, K, axis=1)
    top_s, top_t = pl.pallas_call(
        functools.partial(_merge_body, K * _TOPK),
        out_shape=[
            jax.ShapeDtypeStruct((B, _NUM_DRAFT), jnp.float32),
            jax.ShapeDtypeStruct((B, _NUM_DRAFT), jnp.int32),
        ],
    )(scores.reshape(B, K * _TOPK), tokens.reshape(B, K * _TOPK), prev_rep)
    return top_s, top_t


# speedup vs baseline: 1.4787x; 1.4787x over previous
import functools

import jax
import jax.numpy as jnp
from jax import lax
from jax.experimental import pallas as pl
from jax.experimental.pallas import tpu as pltpu
from jax.experimental.pallas import tpu_sc as plsc

_TOPK = 8
_NUM_DRAFT = 48
_NEG_INF = float("-inf")
_BIG_I32 = 2**30
_W = 800
_CW = 4000
_Q = _CW // _W


def _stats_body(K, NC, x_ref, sm_ref, m0_ref, lse_ref):
    x = x_ref[0]
    pieces = []
    for q in range(_Q):
        xq = x[:, :, q * _W:(q + 1) * _W]
        pieces.append(jnp.max(xq, axis=2))
    sm = jnp.concatenate(pieces, axis=1)
    m0 = jnp.max(sm, axis=1, keepdims=True)
    se = jnp.sum(jnp.exp(x - m0[:, :, None]), axis=2)
    lse = jnp.log(jnp.sum(se, axis=1, keepdims=True))
    sm_ref[0] = sm
    m0_ref[0] = m0
    lse_ref[0] = lse


def _strip_select_body(S, NC, sm_ref, voff_ref, flat_ref):
    sm = sm_ref[...]
    R = sm.shape[0]
    lane_s = lax.broadcasted_iota(jnp.int32, (R, S), 1)
    q = lane_s // NC
    c = lane_s - q * NC
    voff = c * _CW + q * _W
    lane_k = lax.broadcasted_iota(jnp.int32, (R, _TOPK), 1)
    keep = jnp.zeros((R, S), jnp.bool_)
    for _ in range(_TOPK):
        m = jnp.max(sm, axis=1, keepdims=True)
        vsel = jnp.min(jnp.where(sm == m, voff, _BIG_I32), axis=1,
                       keepdims=True)
        hit = voff == vsel
        keep = jnp.logical_or(keep, hit)
        sm = jnp.where(hit, _NEG_INF, sm)
    voffs = jnp.zeros((R, _TOPK), jnp.int32)
    for k in range(_TOPK):
        vo_k = jnp.min(jnp.where(keep, voff, _BIG_I32), axis=1,
                       keepdims=True)
        keep = jnp.logical_and(keep, voff != vo_k)
        voffs = jnp.where(lane_k == k, vo_k, voffs)
    voff_ref[...] = voffs
    row_iota = lax.broadcasted_iota(jnp.int32, (R, _TOPK), 0)
    flat_ref[...] = row_iota * S + voffs // _W


def _make_sc_gather(n_idx, W, n_workers):
    chunk = n_idx // n_workers
    mesh = plsc.VectorSubcoreMesh(core_axis_name="c", subcore_axis_name="s")

    @functools.partial(
        pl.kernel,
        mesh=mesh,
        compiler_params=pltpu.CompilerParams(use_tc_tiling_on_sc=False),
        out_type=jax.ShapeDtypeStruct((n_idx, W), jnp.float32),
        scratch_types=[
            pltpu.VMEM((chunk,), jnp.int32),
            pltpu.VMEM((chunk, W), jnp.float32),
            pltpu.SemaphoreType.DMA,
        ],
    )
    def gather_sc(idx_hbm, table_hbm, out_hbm, idx_v, rows_v, sem):
        info = plsc.get_sparse_core_info()
        wid = lax.axis_index("s") * info.num_cores + lax.axis_index("c")
        base = wid * chunk
        pltpu.sync_copy(idx_hbm.at[pl.ds(base, chunk)], idx_v)
        pltpu.async_copy(table_hbm.at[idx_v], rows_v, sem).wait()
        pltpu.sync_copy(rows_v, out_hbm.at[pl.ds(base, chunk)])

    return gather_sc


def _row_topk_body(W, g_ref, voff_ref, m0_ref, lse_ref, scores_ref,
                   tokens_ref):
    g = g_ref[...]
    voffs = voff_ref[...]
    RB = g.shape[0]
    lane = lax.broadcasted_iota(jnp.int32, (1, 1, W), 2)
    gi = voffs[:, :, None] + lane
    lane_k = lax.broadcasted_iota(jnp.int32, (RB, _TOPK), 1)
    vals = jnp.zeros((RB, _TOPK), jnp.float32)
    toks = jnp.zeros((RB, _TOPK), jnp.int32)
    for i in range(_TOPK):
        m = jnp.max(g, axis=(1, 2), keepdims=True)
        jv = jnp.min(jnp.where(g == m, gi, _BIG_I32), axis=(1, 2),
                     keepdims=True)
        vals = jnp.where(lane_k == i, m[:, 0, :], vals)
        toks = jnp.where(lane_k == i, jv[:, 0, :], toks)
        g = jnp.where(gi == jv, _NEG_INF, g)
    scores_ref[...] = (vals - m0_ref[...]) - lse_ref[...]
    tokens_ref[...] = toks


def _merge_body(KK, scores_ref, tokens_ref, prev_ref, out_s_ref, out_t_ref):
    cum = scores_ref[...] + prev_ref[...]
    toks = tokens_ref[...]
    B = cum.shape[0]
    lane_kk = lax.broadcasted_iota(jnp.int32, (B, KK), 1)
    lane_t = lax.broadcasted_iota(jnp.int32, (B, _NUM_DRAFT), 1)

    def body(i, carry):
        cum, outv, outt = carry
        m = jnp.max(cum, axis=1, keepdims=True)
        jsel = jnp.min(jnp.where(cum == m, lane_kk, _BIG_I32), axis=1,
                       keepdims=True)
        tok = jnp.min(jnp.where(lane_kk == jsel, toks, _BIG_I32), axis=1,
                      keepdims=True)
        outv = jnp.where(lane_t == i, m, outv)
        outt = jnp.where(lane_t == i, tok, outt)
        cum = jnp.where(lane_kk == jsel, _NEG_INF, cum)
        return cum, outv, outt

    outv = jnp.zeros((B, _NUM_DRAFT), jnp.float32)
    outt = jnp.zeros((B, _NUM_DRAFT), jnp.int32)
    _, outv, outt = lax.fori_loop(0, _NUM_DRAFT, body, (cum, outv, outt))
    out_s_ref[...] = outv
    out_t_ref[...] = outt


def kernel(logits, prev_scores):
    B, K, V = logits.shape
    NC = V // _CW
    S = V // _W
    R = B * K

    x4 = logits.reshape(B, K, NC, _CW)
    sm, m0, lse = pl.pallas_call(
        functools.partial(_stats_body, K, NC),
        grid=(B,),
        in_specs=[pl.BlockSpec((1, K, NC, _CW), lambda i: (i, 0, 0, 0))],
        out_specs=[
            pl.BlockSpec((1, K, S), lambda i: (i, 0, 0)),
            pl.BlockSpec((1, K, 1), lambda i: (i, 0, 0)),
            pl.BlockSpec((1, K, 1), lambda i: (i, 0, 0)),
        ],
        out_shape=[
            jax.ShapeDtypeStruct((B, K, S), jnp.float32),
            jax.ShapeDtypeStruct((B, K, 1), jnp.float32),
            jax.ShapeDtypeStruct((B, K, 1), jnp.float32),
        ],
    )(x4)

    voffs, flat_ids = pl.pallas_call(
        functools.partial(_strip_select_body, S, NC),
        out_shape=[
            jax.ShapeDtypeStruct((R, _TOPK), jnp.int32),
            jax.ShapeDtypeStruct((R, _TOPK), jnp.int32),
        ],
    )(sm.reshape(R, S))

    n_idx = R * _TOPK
    gathered = _make_sc_gather(n_idx, _W, 32)(
        flat_ids.reshape(n_idx), logits.reshape(R * S, _W))

    RB = min(64, R)
    scores, tokens = pl.pallas_call(
        functools.partial(_row_topk_body, _W),
        grid=(R // RB,),
        in_specs=[
            pl.BlockSpec((RB, _TOPK, _W), lambda i: (i, 0, 0)),
            pl.BlockSpec((RB, _TOPK), lambda i: (i, 0)),
            pl.BlockSpec((RB, 1), lambda i: (i, 0)),
            pl.BlockSpec((RB, 1), lambda i: (i, 0)),
        ],
        out_specs=[
            pl.BlockSpec((RB, _TOPK), lambda i: (i, 0)),
            pl.BlockSpec((RB, _TOPK), lambda i: (i, 0)),
        ],
        out_shape=[
            jax.ShapeDtypeStruct((R, _TOPK), jnp.float32),
            jax.ShapeDtypeStruct((R, _TOPK), jnp.int32),
        ],
    )(gathered.reshape(R, _TOPK, _W), voffs, m0.reshape(R, 1),
      lse.reshape(R, 1))

    prev_rep = jnp.repeat(prev_scores, K, axis=1)
    top_s, top_t = pl.pallas_call(
        functools.partial(_merge_body, K * _TOPK),
        out_shape=[
            jax.ShapeDtypeStruct((B, _NUM_DRAFT), jnp.float32),
            jax.ShapeDtypeStruct((B, _NUM_DRAFT), jnp.int32),
        ],
    )(scores.reshape(B, K * _TOPK), tokens.reshape(B, K * _TOPK), prev_rep)
    return top_s, top_t


# speedup vs baseline: 2.0485x; 1.3854x over previous
import functools

import jax
import jax.numpy as jnp
from jax import lax
from jax.experimental import pallas as pl
from jax.experimental.pallas import tpu as pltpu
from jax.experimental.pallas import tpu_sc as plsc

_TOPK = 8
_NUM_DRAFT = 48
_NEG_INF = float("-inf")
_BIG_I32 = 2**30
_W = 800
_CW = 4000
_Q = _CW // _W


def _stats_body(K, S, x_ref, sm_ref, m0_ref, lse_ref):
    x = x_ref[0]
    pieces = []
    for s in range(S):
        xq = x[:, s * _W:(s + 1) * _W]
        pieces.append(jnp.max(xq, axis=1, keepdims=True))
    sm = jnp.concatenate(pieces, axis=1)
    m0 = jnp.max(sm, axis=1, keepdims=True)
    lse = jnp.log(jnp.sum(jnp.exp(x - m0), axis=1, keepdims=True))
    sm_ref[0] = sm
    m0_ref[0] = m0
    lse_ref[0] = lse


def _strip_select_body(S, NC, sm_ref, voff_ref, flat_ref):
    sm = sm_ref[...]
    R = sm.shape[0]
    lane_s = lax.broadcasted_iota(jnp.int32, (R, S), 1)
    q = lane_s // NC
    c = lane_s - q * NC
    voff = c * _CW + q * _W
    lane_k = lax.broadcasted_iota(jnp.int32, (R, _TOPK), 1)
    keep = jnp.zeros((R, S), jnp.bool_)
    for _ in range(_TOPK):
        m = jnp.max(sm, axis=1, keepdims=True)
        vsel = jnp.min(jnp.where(sm == m, voff, _BIG_I32), axis=1,
                       keepdims=True)
        hit = voff == vsel
        keep = jnp.logical_or(keep, hit)
        sm = jnp.where(hit, _NEG_INF, sm)
    voffs = jnp.zeros((R, _TOPK), jnp.int32)
    for k in range(_TOPK):
        vo_k = jnp.min(jnp.where(keep, voff, _BIG_I32), axis=1,
                       keepdims=True)
        keep = jnp.logical_and(keep, voff != vo_k)
        voffs = jnp.where(lane_k == k, vo_k, voffs)
    voff_ref[...] = voffs
    row_iota = lax.broadcasted_iota(jnp.int32, (R, _TOPK), 0)
    flat_ref[...] = row_iota * S + voffs // _W


def _make_sc_gather(n_idx, W, n_workers):
    chunk = n_idx // n_workers
    mesh = plsc.VectorSubcoreMesh(core_axis_name="c", subcore_axis_name="s")

    @functools.partial(
        pl.kernel,
        mesh=mesh,
        compiler_params=pltpu.CompilerParams(use_tc_tiling_on_sc=False),
        out_type=jax.ShapeDtypeStruct((n_idx, W), jnp.float32),
        scratch_types=[
            pltpu.VMEM((chunk,), jnp.int32),
            pltpu.VMEM((chunk, W), jnp.float32),
            pltpu.SemaphoreType.DMA,
        ],
    )
    def gather_sc(idx_hbm, table_hbm, out_hbm, idx_v, rows_v, sem):
        info = plsc.get_sparse_core_info()
        wid = lax.axis_index("s") * info.num_cores + lax.axis_index("c")
        base = wid * chunk
        pltpu.sync_copy(idx_hbm.at[pl.ds(base, chunk)], idx_v)
        pltpu.async_copy(table_hbm.at[idx_v], rows_v, sem).wait()
        pltpu.sync_copy(rows_v, out_hbm.at[pl.ds(base, chunk)])

    return gather_sc


def _row_topk_body(W, g_ref, voff_ref, m0_ref, lse_ref, scores_ref,
                   tokens_ref):
    g = g_ref[...]
    voffs = voff_ref[...]
    RB = g.shape[0]
    lane = lax.broadcasted_iota(jnp.int32, (1, 1, W), 2)
    gi = voffs[:, :, None] + lane
    lane_k = lax.broadcasted_iota(jnp.int32, (RB, _TOPK), 1)
    vals = jnp.zeros((RB, _TOPK), jnp.float32)
    toks = jnp.zeros((RB, _TOPK), jnp.int32)
    for i in range(_TOPK):
        m = jnp.max(g, axis=(1, 2), keepdims=True)
        jv = jnp.min(jnp.where(g == m, gi, _BIG_I32), axis=(1, 2),
                     keepdims=True)
        vals = jnp.where(lane_k == i, m[:, 0, :], vals)
        toks = jnp.where(lane_k == i, jv[:, 0, :], toks)
        g = jnp.where(gi == jv, _NEG_INF, g)
    scores_ref[...] = (vals - m0_ref[...]) - lse_ref[...]
    tokens_ref[...] = toks


def _merge_body(KK, scores_ref, tokens_ref, prev_ref, out_s_ref, out_t_ref):
    cum = scores_ref[...] + prev_ref[...]
    toks = tokens_ref[...]
    B = cum.shape[0]
    lane_kk = lax.broadcasted_iota(jnp.int32, (B, KK), 1)
    lane_t = lax.broadcasted_iota(jnp.int32, (B, _NUM_DRAFT), 1)

    def body(i, carry):
        cum, outv, outt = carry
        m = jnp.max(cum, axis=1, keepdims=True)
        jsel = jnp.min(jnp.where(cum == m, lane_kk, _BIG_I32), axis=1,
                       keepdims=True)
        tok = jnp.min(jnp.where(lane_kk == jsel, toks, _BIG_I32), axis=1,
                      keepdims=True)
        outv = jnp.where(lane_t == i, m, outv)
        outt = jnp.where(lane_t == i, tok, outt)
        cum = jnp.where(lane_kk == jsel, _NEG_INF, cum)
        return cum, outv, outt

    outv = jnp.zeros((B, _NUM_DRAFT), jnp.float32)
    outt = jnp.zeros((B, _NUM_DRAFT), jnp.int32)
    _, outv, outt = lax.fori_loop(0, _NUM_DRAFT, body, (cum, outv, outt))
    out_s_ref[...] = outv
    out_t_ref[...] = outt


def kernel(logits, prev_scores):
    B, K, V = logits.shape
    NC = V // _CW
    S = V // _W
    R = B * K

    x4 = logits
    sm, m0, lse = pl.pallas_call(
        functools.partial(_stats_body, K, S),
        grid=(B,),
        in_specs=[pl.BlockSpec((1, K, V), lambda i: (i, 0, 0))],
        out_specs=[
            pl.BlockSpec((1, K, S), lambda i: (i, 0, 0)),
            pl.BlockSpec((1, K, 1), lambda i: (i, 0, 0)),
            pl.BlockSpec((1, K, 1), lambda i: (i, 0, 0)),
        ],
        out_shape=[
            jax.ShapeDtypeStruct((B, K, S), jnp.float32),
            jax.ShapeDtypeStruct((B, K, 1), jnp.float32),
            jax.ShapeDtypeStruct((B, K, 1), jnp.float32),
        ],
    )(x4)

    voffs, flat_ids = pl.pallas_call(
        functools.partial(_strip_select_body, S, 1),
        out_shape=[
            jax.ShapeDtypeStruct((R, _TOPK), jnp.int32),
            jax.ShapeDtypeStruct((R, _TOPK), jnp.int32),
        ],
    )(sm.reshape(R, S))

    n_idx = R * _TOPK
    gathered = _make_sc_gather(n_idx, _W, 32)(
        flat_ids.reshape(n_idx), logits.reshape(R * S, _W))

    RB = min(64, R)
    scores, tokens = pl.pallas_call(
        functools.partial(_row_topk_body, _W),
        grid=(R // RB,),
        in_specs=[
            pl.BlockSpec((RB, _TOPK, _W), lambda i: (i, 0, 0)),
            pl.BlockSpec((RB, _TOPK), lambda i: (i, 0)),
            pl.BlockSpec((RB, 1), lambda i: (i, 0)),
            pl.BlockSpec((RB, 1), lambda i: (i, 0)),
        ],
        out_specs=[
            pl.BlockSpec((RB, _TOPK), lambda i: (i, 0)),
            pl.BlockSpec((RB, _TOPK), lambda i: (i, 0)),
        ],
        out_shape=[
            jax.ShapeDtypeStruct((R, _TOPK), jnp.float32),
            jax.ShapeDtypeStruct((R, _TOPK), jnp.int32),
        ],
    )(gathered.reshape(R, _TOPK, _W), voffs, m0.reshape(R, 1),
      lse.reshape(R, 1))

    prev_rep = jnp.repeat(prev_scores, K, axis=1)
    top_s, top_t = pl.pallas_call(
        functools.partial(_merge_body, K * _TOPK),
        out_shape=[
            jax.ShapeDtypeStruct((B, _NUM_DRAFT), jnp.float32),
            jax.ShapeDtypeStruct((B, _NUM_DRAFT), jnp.int32),
        ],
    )(scores.reshape(B, K * _TOPK), tokens.reshape(B, K * _TOPK), prev_rep)
    return top_s, top_t
